# Initial kernel scaffold; baseline (speedup 1.0000x reference)
import jax
import jax.numpy as jnp
from jax.experimental import pallas as pl


def kernel(x, c_lin_w, c_lin_b, v_lin_w, v_lin_b):
    raise NotImplementedError("write your pallas kernel here")


# speedup vs baseline: 1.1949x; 1.1949x over previous
import functools

import jax
import jax.numpy as jnp
import numpy as np
from jax.experimental import pallas as pl

_N = 64
_POOLING_COUNTS = [15, 8, 8]
_D_TILE = 128


def _build_mask() -> np.ndarray:
    mask = np.zeros((_N, _N), dtype=bool)
    d = np.arange(_N)
    mask[d, d] = True
    stride, offset = 1, 0
    for c in _POOLING_COUNTS:
        for _ in range(c):
            offset += stride
            i = np.arange(0, _N - offset, stride)
            j = np.arange(offset, _N, stride)
            mask[i, j] = True
        stride *= 2
    return mask


_MASK_NP = _build_mask()

def _iota_ij():
    i_idx = jax.lax.broadcasted_iota(jnp.int32, (_N, _N), 0)
    j_idx = jax.lax.broadcasted_iota(jnp.int32, (_N, _N), 1)
    return i_idx, j_idx


def _mask_from_iota():
    i_idx, j_idx = _iota_ij()
    o = j_idx - i_idx
    g0 = (o >= 0) & (o <= 15)
    g1 = (o >= 17) & (o <= 31) & (o % 2 == 1) & (i_idx % 2 == 0)
    g2 = (o >= 35) & (o <= 63) & (o % 4 == 3) & (i_idx % 4 == 0)
    return (g0 | g1 | g2).astype(jnp.float32)


def _attn_kernel(x_ref, qwt_ref, qb_ref, vwt_ref, vb_ref, f_ref):
    xb = x_ref[0]
    xt = xb.T
    m_k = jnp.dot(xt, vwt_ref[...], preferred_element_type=jnp.float32)
    m_k = m_k + vb_ref[0][None, :]
    m_q = jnp.dot(xt, qwt_ref[...], preferred_element_type=jnp.float32)
    m_q = m_q + qb_ref[0][None, :]
    m2m = jax.lax.dot_general(
        m_k, m_q, (((1,), (1,)), ((), ())),
        preferred_element_type=jnp.float32) * 0.125
    m2m = m2m - jnp.max(m2m, axis=-1, keepdims=True)
    e = jnp.exp(m2m)
    w = e / jnp.sum(e, axis=-1, keepdims=True)
    f_ref[0] = _mask_from_iota() * (1.0 + w)


def _maps_kernel(x_ref, f_ref, b_ref, l_ref, c_ref):
    xb = x_ref[0]
    f = f_ref[0]
    xi = xb[:, :, None]
    xj = xb[:, None, :]
    s = xi + xj
    b_ref[0] = s * (0.5 * f)

    i_idx, j_idx = _iota_ij()
    rep_m = jax.lax.broadcasted_iota(jnp.int32, (_N, 2 * _N), 0)
    rep_l = jax.lax.broadcasted_iota(jnp.int32, (_N, 2 * _N), 1)
    rep_oh = (rep_l // 2 == rep_m).astype(jnp.float32)
    rep2 = jnp.dot(xb, rep_oh, preferred_element_type=jnp.float32)
    h = jnp.broadcast_to(rep2[:, None, :], (xb.shape[0], _N, 2 * _N))
    row128 = jax.lax.broadcasted_iota(jnp.int32, (_N, 2 * _N), 0)
    for k in range(6):
        rolled = jnp.roll(h, -(1 << k), axis=-1)
        h = jnp.where((row128 & (1 << k)) != 0, rolled, h)
    a = h[:, :, :_N]
    l_ref[0] = (s + 0.5 * a) * (f * (1.0 / 2.5))

    m = jnp.broadcast_to(xj, s.shape)
    for k in range(6):
        rolled = jnp.roll(m, 1 << k, axis=-1)
        m = jnp.where(j_idx - i_idx >= (1 << k),
                      jnp.maximum(m, rolled), m)
    c_ref[0] = m * f


def kernel(x, c_lin_w, c_lin_b, v_lin_w, v_lin_b):
    bsz, dim, n = x.shape
    odim = v_lin_w.shape[0]
    qwt = c_lin_w[:odim].T
    qb = c_lin_b[:odim].reshape(1, odim)
    vwt = v_lin_w.T
    vb = v_lin_b.reshape(1, odim)

    f = pl.pallas_call(
        _attn_kernel,
        grid=(bsz,),
        in_specs=[
            pl.BlockSpec((1, dim, n), lambda b: (b, 0, 0)),
            pl.BlockSpec((dim, odim), lambda b: (0, 0)),
            pl.BlockSpec((1, odim), lambda b: (0, 0)),
            pl.BlockSpec((dim, odim), lambda b: (0, 0)),
            pl.BlockSpec((1, odim), lambda b: (0, 0)),
        ],
        out_specs=pl.BlockSpec((1, n, n), lambda b: (b, 0, 0)),
        out_shape=jax.ShapeDtypeStruct((bsz, n, n), jnp.float32),
    )(x, qwt, qb, vwt, vb)

    dt = _D_TILE
    maps = pl.pallas_call(
        _maps_kernel,
        grid=(bsz, dim // dt),
        in_specs=[
            pl.BlockSpec((1, dt, n), lambda b, d: (b, d, 0)),
            pl.BlockSpec((1, n, n), lambda b, d: (b, 0, 0)),
        ],
        out_specs=[
            pl.BlockSpec((1, dt, n, n), lambda b, d: (b, d, 0, 0)),
            pl.BlockSpec((1, dt, n, n), lambda b, d: (b, d, 0, 0)),
            pl.BlockSpec((1, dt, n, n), lambda b, d: (b, d, 0, 0)),
        ],
        out_shape=[
            jax.ShapeDtypeStruct((bsz, dim, n, n), jnp.float32),
            jax.ShapeDtypeStruct((bsz, dim, n, n), jnp.float32),
            jax.ShapeDtypeStruct((bsz, dim, n, n), jnp.float32),
        ],
    )(x, f)
    b_map, l_map, c_map = maps

    mask2d = jnp.broadcast_to(
        jnp.asarray(_MASK_NP)[None, None, :, :], (bsz, 1, n, n))
    return (b_map, l_map, c_map, mask2d)


# speedup vs baseline: 3.0415x; 2.5455x over previous
import jax
import jax.numpy as jnp
import numpy as np
from jax.experimental import pallas as pl

_N = 64
_NSQ = _N * _N
_POOLING_COUNTS = [15, 8, 8]
_D_TILE = 128


def _build_mask() -> np.ndarray:
    mask = np.zeros((_N, _N), dtype=bool)
    d = np.arange(_N)
    mask[d, d] = True
    stride, offset = 1, 0
    for c in _POOLING_COUNTS:
        for _ in range(c):
            offset += stride
            i = np.arange(0, _N - offset, stride)
            j = np.arange(offset, _N, stride)
            mask[i, j] = True
        stride *= 2
    return mask


_MASK_NP = _build_mask()

_MM = np.arange(_N)[:, None]
_QI = (np.arange(_NSQ) // _N)[None, :]
_QJ = (np.arange(_NSQ) % _N)[None, :]
_QMID = (_QI + _QJ) // 2
_EQI = (_MM == _QI).astype(np.float32)
_EQJ = (_MM == _QJ).astype(np.float32)
_EQM = (_MM == _QMID).astype(np.float32)
_OH_B = (0.5 * (_EQI + _EQJ)).astype(np.float32)
_OH_L = (0.4 * (_EQI + _EQJ) + 0.2 * _EQM).astype(np.float32)
_OH_J = _EQJ


def _mask_from_iota():
    i_idx = jax.lax.broadcasted_iota(jnp.int32, (_N, _N), 0)
    j_idx = jax.lax.broadcasted_iota(jnp.int32, (_N, _N), 1)
    o = j_idx - i_idx
    g0 = (o >= 0) & (o <= 15)
    g1 = (o >= 17) & (o <= 31) & (o % 2 == 1) & (i_idx % 2 == 0)
    g2 = (o >= 35) & (o <= 63) & (o % 4 == 3) & (i_idx % 4 == 0)
    return (g0 | g1 | g2).astype(jnp.float32)


def _attn_kernel(x_ref, qwt_ref, qb_ref, vwt_ref, vb_ref, f_ref):
    xb = x_ref[0]
    xt = xb.T
    m_k = jnp.dot(xt, vwt_ref[...], preferred_element_type=jnp.float32)
    m_k = m_k + vb_ref[0][None, :]
    m_q = jnp.dot(xt, qwt_ref[...], preferred_element_type=jnp.float32)
    m_q = m_q + qb_ref[0][None, :]
    m2m = jax.lax.dot_general(
        m_k, m_q, (((1,), (1,)), ((), ())),
        preferred_element_type=jnp.float32) * 0.125
    m2m = m2m - jnp.max(m2m, axis=-1, keepdims=True)
    e = jnp.exp(m2m)
    w = e / jnp.sum(e, axis=-1, keepdims=True)
    f_ref[...] = (_mask_from_iota() * (1.0 + w))[None, None]


def _maps_kernel(x_ref, f_ref, ohb_ref, ohl_ref, ohj_ref,
                 b_ref, l_ref, c_ref):
    xb = x_ref[0]
    f = f_ref[0]

    b_ref[0] = jnp.dot(xb, ohb_ref[...],
                       preferred_element_type=jnp.float32) * f
    l_ref[0] = jnp.dot(xb, ohl_ref[...],
                       preferred_element_type=jnp.float32) * f

    q = jax.lax.broadcasted_iota(jnp.int32, (1, _NSQ), 1)
    diag = q % _N - q // _N
    m = jnp.dot(xb, ohj_ref[...], preferred_element_type=jnp.float32)
    for k in range(6):
        rolled = jnp.roll(m, 1 << k, axis=-1)
        m = jnp.where(diag >= (1 << k), jnp.maximum(m, rolled), m)
    c_ref[0] = m * f


def kernel(x, c_lin_w, c_lin_b, v_lin_w, v_lin_b):
    bsz, dim, n = x.shape
    odim = v_lin_w.shape[0]
    qwt = c_lin_w[:odim].T
    qb = c_lin_b[:odim].reshape(1, odim)
    vwt = v_lin_w.T
    vb = v_lin_b.reshape(1, odim)

    f = pl.pallas_call(
        _attn_kernel,
        grid=(bsz,),
        in_specs=[
            pl.BlockSpec((1, dim, n), lambda b: (b, 0, 0)),
            pl.BlockSpec((dim, odim), lambda b: (0, 0)),
            pl.BlockSpec((1, odim), lambda b: (0, 0)),
            pl.BlockSpec((dim, odim), lambda b: (0, 0)),
            pl.BlockSpec((1, odim), lambda b: (0, 0)),
        ],
        out_specs=pl.BlockSpec((1, 1, n, n), lambda b: (b, 0, 0, 0)),
        out_shape=jax.ShapeDtypeStruct((bsz, 1, n, n), jnp.float32),
    )(x, qwt, qb, vwt, vb)
    f = f.reshape(bsz, 1, _NSQ)

    dt = _D_TILE
    oh_b = jnp.asarray(_OH_B)
    oh_l = jnp.asarray(_OH_L)
    oh_j = jnp.asarray(_OH_J)
    maps = pl.pallas_call(
        _maps_kernel,
        grid=(bsz, dim // dt),
        in_specs=[
            pl.BlockSpec((1, dt, n), lambda b, d: (b, d, 0)),
            pl.BlockSpec((1, 1, _NSQ), lambda b, d: (b, 0, 0)),
            pl.BlockSpec((n, _NSQ), lambda b, d: (0, 0)),
            pl.BlockSpec((n, _NSQ), lambda b, d: (0, 0)),
            pl.BlockSpec((n, _NSQ), lambda b, d: (0, 0)),
        ],
        out_specs=[
            pl.BlockSpec((1, dt, _NSQ), lambda b, d: (b, d, 0)),
            pl.BlockSpec((1, dt, _NSQ), lambda b, d: (b, d, 0)),
            pl.BlockSpec((1, dt, _NSQ), lambda b, d: (b, d, 0)),
        ],
        out_shape=[
            jax.ShapeDtypeStruct((bsz, dim, _NSQ), jnp.float32),
            jax.ShapeDtypeStruct((bsz, dim, _NSQ), jnp.float32),
            jax.ShapeDtypeStruct((bsz, dim, _NSQ), jnp.float32),
        ],
    )(x, f, oh_b, oh_l, oh_j)
    b_map = maps[0].reshape(bsz, dim, n, n)
    l_map = maps[1].reshape(bsz, dim, n, n)
    c_map = maps[2].reshape(bsz, dim, n, n)

    mask2d = jnp.broadcast_to(
        jnp.asarray(_MASK_NP)[None, None, :, :], (bsz, 1, n, n))
    return (b_map, l_map, c_map, mask2d)
